# baseline (speedup 1.0000x reference)
import jax
import jax.numpy as jnp
from jax.experimental import pallas as pl
from jax.experimental.pallas import tpu as pltpu

S = 2048
DIM = 768
H = 12
DH = 64
BLK = 4
NB = S // BLK
WIN = 32
QT = 256
NT = S // QT
SCALE = DH ** -0.5
NEG = -1e9
BF = jnp.bfloat16
F32 = jnp.float32


def _prep_body(lam_ref, x_ref, x0_ref, wq_ref, wk_ref, wv_ref, wg_ref,
               wkc_ref, wvc_ref, pek_ref, pev_ref,
               xa_ref, q_ref, k_ref, v_ref, g_ref, ck_ref, cv_ref):
    lam0 = lam_ref[0]
    lam1 = lam_ref[1]
    xa = lam0 * x_ref[...] + lam1 * x0_ref[...]
    xa_ref[...] = xa
    h = xa * jax.lax.rsqrt(jnp.mean(xa * xa, axis=-1, keepdims=True) + 1e-6)
    hb = h.astype(BF)
    q = jnp.dot(hb, wq_ref[...], preferred_element_type=F32)
    k = jnp.dot(hb, wk_ref[...], preferred_element_type=F32)
    v = jnp.dot(hb, wv_ref[...], preferred_element_type=F32)
    q_ref[...] = q.astype(BF)
    kb = k.astype(BF)
    vb = v.astype(BF)
    k_ref[...] = kb
    v_ref[...] = vb
    g_ref[...] = jax.nn.sigmoid(
        jnp.dot(hb, wg_ref[...], preferred_element_type=F32))
    k4 = kb.reshape(QT // BLK, BLK, DIM)
    v4 = vb.reshape(QT // BLK, BLK, DIM)
    ck = jnp.broadcast_to(pek_ref[...], (QT // BLK, DIM))
    cv = jnp.broadcast_to(pev_ref[...], (QT // BLK, DIM))
    for r in range(BLK):
        ck = ck + jnp.dot(k4[:, r, :], wkc_ref[r], preferred_element_type=F32)
        cv = cv + jnp.dot(v4[:, r, :], wvc_ref[r], preferred_element_type=F32)
    ck_ref[...] = ck.astype(BF)
    cv_ref[...] = cv.astype(BF)


SW = QT + 32


def _attn_body(q_ref, k_ref, v_ref, ck_ref, cv_ref, g_ref, fm_ref, o_ref):
    t = pl.program_id(0)
    row = t * QT + jax.lax.broadcasted_iota(jnp.int32, (QT, 1), 0)
    soff = pl.multiple_of(jnp.maximum(t * QT - 32, 0), 32)
    scol = soff + jax.lax.broadcasted_iota(jnp.int32, (QT, SW), 1)
    d = row - scol
    smask = (d >= 0) & (d < WIN)
    cpb = QT // BLK
    cblk0 = jax.lax.broadcasted_iota(jnp.int32, (QT, cpb), 1)

    contract = (((1,), (1,)), ((), ()))
    for h in range(H):
        sl = slice(h * DH, (h + 1) * DH)
        q_h = q_ref[:, sl]

        def body(kt, carry):
            mf, lf, af, mc, lc, ac = carry
            k_c = k_ref[pl.ds(kt * QT, QT), sl]
            v_c = v_ref[pl.ds(kt * QT, QT), sl]
            sim = jax.lax.dot_general(q_h, k_c, contract,
                                      preferred_element_type=F32) * SCALE
            fm_c = fm_ref[:, pl.ds(kt * QT, QT)] != 0
            fsim = jnp.where(fm_c, sim, NEG)
            mf_n = jnp.maximum(mf, jnp.max(fsim, axis=-1, keepdims=True))
            fp = jnp.where(fm_c, jnp.exp(fsim - mf_n), 0.0)
            fscale = jnp.exp(mf - mf_n)
            lf_n = lf * fscale + jnp.sum(fp, axis=-1, keepdims=True)
            af_n = af * fscale + jnp.dot(fp.astype(BF), v_c,
                                         preferred_element_type=F32)

            ck_c = ck_ref[pl.ds(kt * cpb, cpb), sl]
            cv_c = cv_ref[pl.ds(kt * cpb, cpb), sl]
            csim = jax.lax.dot_general(q_h, ck_c, contract,
                                       preferred_element_type=F32) * SCALE
            cm_c = (BLK * (kt * cpb + cblk0) + (BLK - 1)) <= row
            csim = jnp.where(cm_c, csim, NEG)
            mc_n = jnp.maximum(mc, jnp.max(csim, axis=-1, keepdims=True))
            cp = jnp.where(cm_c, jnp.exp(csim - mc_n), 0.0)
            cscale = jnp.exp(mc - mc_n)
            lc_n = lc * cscale + jnp.sum(cp, axis=-1, keepdims=True)
            ac_n = ac * cscale + jnp.dot(cp.astype(BF), cv_c,
                                         preferred_element_type=F32)
            return mf_n, lf_n, af_n, mc_n, lc_n, ac_n

        neg = jnp.full((QT, 1), -1e30, dtype=F32)
        zero1 = jnp.zeros((QT, 1), dtype=F32)
        zacc = jnp.zeros((QT, DH), dtype=F32)
        mf, lf, af, mc, lc, ac = jax.lax.fori_loop(
            0, t + 1, body, (neg, zero1, zacc, neg, zero1, zacc))
        f_out = af / lf
        mc_f = jnp.maximum(mc, 0.0)
        cden = lc * jnp.exp(mc - mc_f) + jnp.exp(-mc_f)
        c_out = ac * (jnp.exp(mc - mc_f) / cden)

        k_s = k_ref[pl.ds(soff, SW), sl]
        v_s = v_ref[pl.ds(soff, SW), sl]
        ssim = jax.lax.dot_general(q_h, k_s, contract,
                                   preferred_element_type=F32) * SCALE
        ssim = jnp.where(smask, ssim, NEG)
        smax = jnp.max(ssim, axis=-1, keepdims=True)
        sp = jnp.exp(ssim - smax)
        s_out = jnp.dot(sp.astype(BF), v_s, preferred_element_type=F32)
        s_out = s_out / jnp.sum(sp, axis=-1, keepdims=True)

        gc = g_ref[:, 3 * h:3 * h + 1]
        gf = g_ref[:, 3 * h + 1:3 * h + 2]
        gs = g_ref[:, 3 * h + 2:3 * h + 3]
        o_ref[:, sl] = (gc * c_out + gf * f_out + gs * s_out).astype(BF)


def _out_body(xa_ref, at_ref, wo_ref, wfc_ref, wproj_ref, o_ref):
    x1 = xa_ref[...] + jnp.dot(at_ref[...], wo_ref[...],
                               preferred_element_type=F32)
    h2 = x1 * jax.lax.rsqrt(jnp.mean(x1 * x1, axis=-1, keepdims=True) + 1e-6)
    u = jnp.dot(h2.astype(BF), wfc_ref[...], preferred_element_type=F32)
    u = jnp.square(jnp.maximum(u, 0.0))
    o_ref[...] = x1 + jnp.dot(u.astype(BF), wproj_ref[...],
                              preferred_element_type=F32)


def kernel(x, ve, x0, lambdas, Wq, Wk, Wv, Wo, k_pe, v_pe, Wkc, Wvc, Wg,
           Wfc, Wproj, sliding_window_flex_mask, fine_selection_flex_mask):
    del ve, sliding_window_flex_mask
    x2 = x[0]
    x02 = x0[0]
    eye = jnp.eye(H, dtype=F32)
    wkc_bd = jnp.stack([jnp.kron(eye, Wkc[r * DH:(r + 1) * DH, :])
                        for r in range(BLK)]).astype(BF)
    wvc_bd = jnp.stack([jnp.kron(eye, Wvc[r * DH:(r + 1) * DH, :])
                        for r in range(BLK)]).astype(BF)
    pek = jnp.tile(k_pe.reshape(1, BLK * DH) @ Wkc, (1, H))
    pev = jnp.tile(v_pe.reshape(1, BLK * DH) @ Wvc, (1, H))
    fm8 = fine_selection_flex_mask.astype(jnp.int8)

    tile2 = lambda w: pl.BlockSpec((QT, w), lambda t: (t, 0))
    full = lambda shape: pl.BlockSpec(shape, lambda t: (0,) * len(shape))

    xa, q, k, v, g, ck, cv = pl.pallas_call(
        _prep_body,
        grid=(NT,),
        in_specs=[
            pl.BlockSpec(memory_space=pltpu.SMEM),
            tile2(DIM), tile2(DIM),
            full((DIM, DIM)), full((DIM, DIM)), full((DIM, DIM)),
            full((DIM, 3 * H)),
            full((BLK, DIM, DIM)), full((BLK, DIM, DIM)),
            full((1, DIM)), full((1, DIM)),
        ],
        out_specs=[
            tile2(DIM), tile2(DIM), tile2(DIM), tile2(DIM), tile2(3 * H),
            pl.BlockSpec((QT // BLK, DIM), lambda t: (t, 0)),
            pl.BlockSpec((QT // BLK, DIM), lambda t: (t, 0)),
        ],
        out_shape=[
            jax.ShapeDtypeStruct((S, DIM), F32),
            jax.ShapeDtypeStruct((S, DIM), BF),
            jax.ShapeDtypeStruct((S, DIM), BF),
            jax.ShapeDtypeStruct((S, DIM), BF),
            jax.ShapeDtypeStruct((S, 3 * H), F32),
            jax.ShapeDtypeStruct((NB, DIM), BF),
            jax.ShapeDtypeStruct((NB, DIM), BF),
        ],
    )(lambdas, x2, x02, Wq.astype(BF), Wk.astype(BF), Wv.astype(BF),
      Wg.astype(BF), wkc_bd, wvc_bd, pek, pev)

    at = pl.pallas_call(
        _attn_body,
        grid=(NT,),
        in_specs=[
            tile2(DIM),
            full((S, DIM)), full((S, DIM)),
            full((NB, DIM)), full((NB, DIM)),
            tile2(3 * H),
            tile2(S),
        ],
        out_specs=tile2(DIM),
        out_shape=jax.ShapeDtypeStruct((S, DIM), BF),
    )(q, k, v, ck, cv, g, fm8)

    out = pl.pallas_call(
        _out_body,
        grid=(NT,),
        in_specs=[
            tile2(DIM), tile2(DIM),
            full((DIM, DIM)), full((DIM, 4 * DIM)), full((4 * DIM, DIM)),
        ],
        out_specs=tile2(DIM),
        out_shape=jax.ShapeDtypeStruct((S, DIM), F32),
    )(xa, at, Wo.astype(BF), Wfc.astype(BF), Wproj.astype(BF))

    return out[None]


# speedup vs baseline: 1.7827x; 1.7827x over previous
import jax
import jax.numpy as jnp
from jax.experimental import pallas as pl
from jax.experimental.pallas import tpu as pltpu

S = 2048
DIM = 768
H = 12
DH = 64
BLK = 4
NB = S // BLK
WIN = 32
QT = 256
NT = S // QT
SCALE = DH ** -0.5
NEG = -1e9
BF = jnp.bfloat16
F32 = jnp.float32


def _prep_body(lam_ref, x_ref, x0_ref, wq_ref, wk_ref, wv_ref, wg_ref,
               wkc_ref, wvc_ref, pek_ref, pev_ref,
               xa_ref, q_ref, k_ref, v_ref, g_ref, ck_ref, cv_ref):
    lam0 = lam_ref[0]
    lam1 = lam_ref[1]
    xa = lam0 * x_ref[...] + lam1 * x0_ref[...]
    xa_ref[...] = xa
    h = xa * jax.lax.rsqrt(jnp.mean(xa * xa, axis=-1, keepdims=True) + 1e-6)
    hb = h.astype(BF)
    q = jnp.dot(hb, wq_ref[...], preferred_element_type=F32)
    k = jnp.dot(hb, wk_ref[...], preferred_element_type=F32)
    v = jnp.dot(hb, wv_ref[...], preferred_element_type=F32)
    q_ref[...] = q.astype(BF)
    kb = k.astype(BF)
    vb = v.astype(BF)
    k_ref[...] = kb
    v_ref[...] = vb
    g_ref[...] = jax.nn.sigmoid(
        jnp.dot(hb, wg_ref[...], preferred_element_type=F32))
    k4 = kb.reshape(QT // BLK, BLK, DIM)
    v4 = vb.reshape(QT // BLK, BLK, DIM)
    ck = jnp.broadcast_to(pek_ref[...], (QT // BLK, DIM))
    cv = jnp.broadcast_to(pev_ref[...], (QT // BLK, DIM))
    for r in range(BLK):
        ck = ck + jnp.dot(k4[:, r, :], wkc_ref[r], preferred_element_type=F32)
        cv = cv + jnp.dot(v4[:, r, :], wvc_ref[r], preferred_element_type=F32)
    ck_ref[...] = ck.astype(BF)
    cv_ref[...] = cv.astype(BF)


SW = QT + 32


def _attn_body(q_ref, k_ref, v_ref, ck_ref, cv_ref, g_ref, fm_ref, o_ref):
    t = pl.program_id(0)
    row = t * QT + jax.lax.broadcasted_iota(jnp.int32, (QT, 1), 0)
    soff = pl.multiple_of(jnp.maximum(t * QT - 32, 0), 32)
    scol = soff + jax.lax.broadcasted_iota(jnp.int32, (QT, SW), 1)
    d = row - scol
    smask = (d >= 0) & (d < WIN)
    fmask = fm_ref[...] != 0
    cblk = jax.lax.broadcasted_iota(jnp.int32, (QT, NB), 1)
    cmask = (BLK * cblk + (BLK - 1)) <= row

    contract = (((1,), (1,)), ((), ()))
    for h in range(H):
        sl = slice(h * DH, (h + 1) * DH)
        q_h = q_ref[:, sl]
        k_h = k_ref[:, sl]
        v_h = v_ref[:, sl]

        sim = jax.lax.dot_general(q_h, k_h, contract,
                                  preferred_element_type=F32) * SCALE
        fsim = jnp.where(fmask, sim, NEG)
        fmax = jnp.max(fsim, axis=-1, keepdims=True)
        fp = jnp.exp(fsim - fmax)
        f_out = jnp.dot(fp.astype(BF), v_h, preferred_element_type=F32)
        f_out = f_out / jnp.sum(fp, axis=-1, keepdims=True)

        ck_h = ck_ref[:, sl]
        cv_h = cv_ref[:, sl]
        csim = jax.lax.dot_general(q_h, ck_h, contract,
                                   preferred_element_type=F32) * SCALE
        csim = jnp.where(cmask, csim, NEG)
        cmax = jnp.maximum(jnp.max(csim, axis=-1, keepdims=True), 0.0)
        cp = jnp.exp(csim - cmax)
        cden = jnp.sum(cp, axis=-1, keepdims=True) + jnp.exp(-cmax)
        c_out = jnp.dot(cp.astype(BF), cv_h, preferred_element_type=F32) / cden

        k_s = k_ref[pl.ds(soff, SW), sl]
        v_s = v_ref[pl.ds(soff, SW), sl]
        ssim = jax.lax.dot_general(q_h, k_s, contract,
                                   preferred_element_type=F32) * SCALE
        ssim = jnp.where(smask, ssim, NEG)
        smax = jnp.max(ssim, axis=-1, keepdims=True)
        sp = jnp.exp(ssim - smax)
        s_out = jnp.dot(sp.astype(BF), v_s, preferred_element_type=F32)
        s_out = s_out / jnp.sum(sp, axis=-1, keepdims=True)

        gc = g_ref[:, 3 * h:3 * h + 1]
        gf = g_ref[:, 3 * h + 1:3 * h + 2]
        gs = g_ref[:, 3 * h + 2:3 * h + 3]
        o_ref[:, sl] = (gc * c_out + gf * f_out + gs * s_out).astype(BF)


def _out_body(xa_ref, at_ref, wo_ref, wfc_ref, wproj_ref, o_ref):
    x1 = xa_ref[...] + jnp.dot(at_ref[...], wo_ref[...],
                               preferred_element_type=F32)
    h2 = x1 * jax.lax.rsqrt(jnp.mean(x1 * x1, axis=-1, keepdims=True) + 1e-6)
    u = jnp.dot(h2.astype(BF), wfc_ref[...], preferred_element_type=F32)
    u = jnp.square(jnp.maximum(u, 0.0))
    o_ref[...] = x1 + jnp.dot(u.astype(BF), wproj_ref[...],
                              preferred_element_type=F32)


def kernel(x, ve, x0, lambdas, Wq, Wk, Wv, Wo, k_pe, v_pe, Wkc, Wvc, Wg,
           Wfc, Wproj, sliding_window_flex_mask, fine_selection_flex_mask):
    del ve, sliding_window_flex_mask
    x2 = x[0]
    x02 = x0[0]
    eye = jnp.eye(H, dtype=F32)
    wkc_bd = jnp.stack([jnp.kron(eye, Wkc[r * DH:(r + 1) * DH, :])
                        for r in range(BLK)]).astype(BF)
    wvc_bd = jnp.stack([jnp.kron(eye, Wvc[r * DH:(r + 1) * DH, :])
                        for r in range(BLK)]).astype(BF)
    pek = jnp.tile(k_pe.reshape(1, BLK * DH) @ Wkc, (1, H))
    pev = jnp.tile(v_pe.reshape(1, BLK * DH) @ Wvc, (1, H))
    fm8 = fine_selection_flex_mask.astype(jnp.int8)

    tile2 = lambda w: pl.BlockSpec((QT, w), lambda t: (t, 0))
    full = lambda shape: pl.BlockSpec(shape, lambda t: (0,) * len(shape))

    xa, q, k, v, g, ck, cv = pl.pallas_call(
        _prep_body,
        grid=(NT,),
        in_specs=[
            pl.BlockSpec(memory_space=pltpu.SMEM),
            tile2(DIM), tile2(DIM),
            full((DIM, DIM)), full((DIM, DIM)), full((DIM, DIM)),
            full((DIM, 3 * H)),
            full((BLK, DIM, DIM)), full((BLK, DIM, DIM)),
            full((1, DIM)), full((1, DIM)),
        ],
        out_specs=[
            tile2(DIM), tile2(DIM), tile2(DIM), tile2(DIM), tile2(3 * H),
            pl.BlockSpec((QT // BLK, DIM), lambda t: (t, 0)),
            pl.BlockSpec((QT // BLK, DIM), lambda t: (t, 0)),
        ],
        out_shape=[
            jax.ShapeDtypeStruct((S, DIM), F32),
            jax.ShapeDtypeStruct((S, DIM), BF),
            jax.ShapeDtypeStruct((S, DIM), BF),
            jax.ShapeDtypeStruct((S, DIM), BF),
            jax.ShapeDtypeStruct((S, 3 * H), F32),
            jax.ShapeDtypeStruct((NB, DIM), BF),
            jax.ShapeDtypeStruct((NB, DIM), BF),
        ],
    )(lambdas, x2, x02, Wq.astype(BF), Wk.astype(BF), Wv.astype(BF),
      Wg.astype(BF), wkc_bd, wvc_bd, pek, pev)

    at = pl.pallas_call(
        _attn_body,
        grid=(NT,),
        in_specs=[
            tile2(DIM),
            full((S, DIM)), full((S, DIM)),
            full((NB, DIM)), full((NB, DIM)),
            tile2(3 * H),
            tile2(S),
        ],
        out_specs=tile2(DIM),
        out_shape=jax.ShapeDtypeStruct((S, DIM), BF),
    )(q, k, v, ck, cv, g, fm8)

    out = pl.pallas_call(
        _out_body,
        grid=(NT,),
        in_specs=[
            tile2(DIM), tile2(DIM),
            full((DIM, DIM)), full((DIM, 4 * DIM)), full((4 * DIM, DIM)),
        ],
        out_specs=tile2(DIM),
        out_shape=jax.ShapeDtypeStruct((S, DIM), F32),
    )(xa, at, Wo.astype(BF), Wfc.astype(BF), Wproj.astype(BF))

    return out[None]
